# baseline (speedup 1.0000x reference)
import functools

import jax
import jax.numpy as jnp
from jax import lax
from jax.experimental import pallas as pl
from jax.experimental.pallas import tpu as pltpu
from jax.experimental.pallas import tpu_sc as plsc

VOCAB = 400001
DIM = 300
DIMP = 384
B = 4096 * 50

NUM_CORES = 2
NUM_SUBCORES = 16
NW = NUM_CORES * NUM_SUBCORES
B_PER_W = B // NW
CHUNK = 64
N_CHUNKS = B_PER_W // CHUNK
NFULL = DIM // 16
TAIL = DIM - 16 * NFULL


def _build():
    mesh = plsc.VectorSubcoreMesh(core_axis_name="c", subcore_axis_name="s")

    @functools.partial(
        pl.kernel,
        mesh=mesh,
        compiler_params=pltpu.CompilerParams(needs_layout_passes=False),
        out_type=jax.ShapeDtypeStruct((B, DIM), jnp.float32),
        scratch_types=[
            pltpu.VMEM((B_PER_W,), jnp.int32),
            pltpu.VMEM((2, CHUNK, DIMP), jnp.float32),
            pltpu.VMEM((2, CHUNK, DIM), jnp.float32),
            pltpu.SemaphoreType.DMA,
            pltpu.SemaphoreType.DMA,
            pltpu.SemaphoreType.DMA,
            pltpu.SemaphoreType.DMA,
        ],
    )
    def emb_kernel(
        ids_hbm, table_hbm, out_hbm, idx_v, rows_v, comp_v, g0, g1, o0, o1
    ):
        wid = lax.axis_index("s") * NUM_CORES + lax.axis_index("c")
        base = wid * B_PER_W
        pltpu.sync_copy(ids_hbm.at[pl.ds(base, B_PER_W)], idx_v)
        gsem = (g0, g1)
        osem = (o0, o1)

        iota = lax.iota(jnp.int32, 16)
        pieces = [iota + 16 * k for k in range(NFULL)]
        tail_idx = jnp.minimum(iota + 16 * NFULL, DIM - 1)
        tail_mask = iota < TAIL

        def fire(ci, b):
            pltpu.async_copy(
                table_hbm.at[idx_v.at[pl.ds(ci * CHUNK, CHUNK)]],
                rows_v.at[b],
                gsem[b],
            )

        def drain_gathers(b):
            pltpu.make_async_copy(
                table_hbm.at[pl.ds(0, CHUNK)], rows_v.at[b], gsem[b]
            ).wait()

        def compact_and_write(ci, b):
            bv = lax.broadcast(jnp.int32(b), (16,))

            def row(j):
                jv = lax.broadcast(j, (16,))
                for k in range(NFULL):
                    vals = plsc.load_gather(rows_v, [bv, jv, pieces[k]])
                    plsc.store_scatter(comp_v, [bv, jv, pieces[k]], vals)
                vals = plsc.load_gather(
                    rows_v, [bv, jv, tail_idx], mask=tail_mask
                )
                plsc.store_scatter(
                    comp_v, [bv, jv, tail_idx], vals, mask=tail_mask
                )

            pl.loop(0, CHUNK)(row)
            pltpu.async_copy(
                comp_v.at[b],
                out_hbm.at[pl.ds(base + ci * CHUNK, CHUNK)],
                osem[b],
            )

        def drain_write(b):
            pltpu.make_async_copy(
                out_hbm.at[pl.ds(0, CHUNK)], comp_v.at[b], osem[b]
            ).wait()

        fire(0, 0)
        fire(1, 1)

        def body(h):
            ci0 = 2 * h
            drain_gathers(0)
            compact_and_write(ci0, 0)

            @pl.when(h < N_CHUNKS // 2 - 1)
            def _():
                drain_write(0)
                fire(ci0 + 2, 0)

            drain_gathers(1)
            compact_and_write(ci0 + 1, 1)

            @pl.when(h < N_CHUNKS // 2 - 1)
            def _():
                drain_write(1)
                fire(ci0 + 3, 1)

        pl.loop(0, N_CHUNKS // 2)(body)
        drain_write(0)
        drain_write(1)

    return emb_kernel


_emb = _build()


@jax.jit
def kernel(word_ids, table):
    ids_flat = word_ids.reshape(B).astype(jnp.int32)
    tpad = jnp.pad(table, ((0, 0), (0, DIMP - DIM)))
    out = _emb(ids_flat, tpad)
    return out.reshape(word_ids.shape + (DIM,))


# speedup vs baseline: 2.7610x; 2.7610x over previous
import functools

import jax
import jax.numpy as jnp
from jax import lax
from jax.experimental import pallas as pl
from jax.experimental.pallas import tpu as pltpu
from jax.experimental.pallas import tpu_sc as plsc

VOCAB = 400001
DIM = 300
B = 4096 * 50

NUM_CORES = 2
NUM_SUBCORES = 16
NW = NUM_CORES * NUM_SUBCORES
B_PER_W = B // NW
CHUNK = 128
N_CHUNKS = B_PER_W // CHUNK


def _build():
    mesh = plsc.VectorSubcoreMesh(core_axis_name="c", subcore_axis_name="s")

    @functools.partial(
        pl.kernel,
        mesh=mesh,
        out_type=jax.ShapeDtypeStruct((B, DIM), jnp.float32),
        scratch_types=[
            pltpu.VMEM((B_PER_W,), jnp.int32),
            pltpu.VMEM((2, CHUNK, DIM), jnp.float32),
            pltpu.SemaphoreType.DMA,
            pltpu.SemaphoreType.DMA,
            pltpu.SemaphoreType.DMA,
            pltpu.SemaphoreType.DMA,
        ],
    )
    def emb_kernel(ids_hbm, table_hbm, out_hbm, idx_v, rows_v, g0, g1, o0, o1):
        wid = lax.axis_index("s") * NUM_CORES + lax.axis_index("c")
        base = wid * B_PER_W
        pltpu.sync_copy(ids_hbm.at[pl.ds(base, B_PER_W)], idx_v)
        gsem = (g0, g1)
        osem = (o0, o1)

        def fire(ci, b):
            def vec(v):
                idx16 = idx_v[pl.ds(ci * CHUNK + v * 16, 16)]
                for l in range(16):
                    pltpu.async_copy(
                        table_hbm.at[pl.ds(idx16[l], 1)],
                        rows_v.at[b].at[pl.ds(v * 16 + l, 1)],
                        gsem[b],
                    )

            pl.loop(0, CHUNK // 16)(vec)

        def drain_gathers(b):
            pltpu.make_async_copy(
                out_hbm.at[pl.ds(0, CHUNK)], rows_v.at[b], gsem[b]
            ).wait()

        def drain_write(b):
            pltpu.make_async_copy(
                out_hbm.at[pl.ds(0, CHUNK)], rows_v.at[b], osem[b]
            ).wait()

        def write(ci, b):
            pltpu.async_copy(
                rows_v.at[b], out_hbm.at[pl.ds(base + ci * CHUNK, CHUNK)], osem[b]
            )

        fire(0, 0)

        def body(h):
            ci0 = 2 * h
            pl.when(h >= 1)(lambda: drain_write(1))
            fire(ci0 + 1, 1)
            drain_gathers(0)
            write(ci0, 0)
            @pl.when(h < N_CHUNKS // 2 - 1)
            def _():
                drain_write(0)
                fire(ci0 + 2, 0)

            drain_gathers(1)
            write(ci0 + 1, 1)

        pl.loop(0, N_CHUNKS // 2)(body)
        drain_write(0)
        drain_write(1)

    return emb_kernel


_emb = _build()


@jax.jit
def kernel(word_ids, table):
    ids_flat = word_ids.reshape(B).astype(jnp.int32)
    out = _emb(ids_flat, table)
    return out.reshape(word_ids.shape + (DIM,))
